# Initial kernel scaffold; baseline (speedup 1.0000x reference)
import jax
import jax.numpy as jnp
from jax.experimental import pallas as pl


def kernel(x, fc, bias, W):
    raise NotImplementedError("write your pallas kernel here")


# speedup vs baseline: 10.1148x; 10.1148x over previous
import functools

import jax
import jax.numpy as jnp
import numpy as np
from jax import lax
from jax.experimental import pallas as pl
from jax.experimental.pallas import tpu as pltpu
from jax.experimental.pallas import tpu_sc as plsc

_FIELD_DIM = 4000
NF = 26
D = 16
V = NF * _FIELD_DIM
B = 4096
NC, NS = 2, 16
NW = NC * NS
BPW = B // NW
NB = 4
NCHUNK = BPW // NB
NPAIR = (NF * (NF - 1)) // 2
ROWS_PER_B = 2 * NPAIR
ROWS_PER_CHUNK = ROWS_PER_B * NB
SEG = 128
NSEG = (ROWS_PER_CHUNK + SEG - 1) // SEG
ROWS_PAD = NSEG * SEG

_S = [25 * i - (i * (i - 1)) // 2 for i in range(NF)]

_CL = np.zeros((NF, 32), dtype=np.int32)
for _i in range(NF - 1):
    for _k in range(25 - _i):
        _CL[_i, _k] = (_i + 1 + _k) * V


def _body(idx_hbm, w_hbm, fc_hbm, bias_hbm, cl_hbm, out_hbm,
          idxv, clv, ib, rows, fcib, fcv, biasv, zbuf, sem):
    cid = lax.axis_index("c")
    sid = lax.axis_index("s")
    wid = sid * NC + cid
    base_b = wid * BPW

    pltpu.sync_copy(idx_hbm.at[pl.ds(base_b, BPW), :], idxv)
    pltpu.sync_copy(cl_hbm, clv)
    pltpu.sync_copy(bias_hbm, biasv)

    zero16 = jnp.zeros((16,), jnp.int32)
    o = ROWS_PER_CHUNK
    while o < ROWS_PAD:
        ib[pl.ds(min(o, ROWS_PAD - 16), 16)] = zero16
        o += 16

    lane = lax.iota(jnp.int32, 16)
    fc_mask = lane >= 6
    bias_mask = lane == 0

    def allsum(v):
        return jnp.sum(v)

    @pl.loop(0, NCHUNK)
    def _chunk(c):
        for bb in range(NB):
            b = c * NB + bb
            lbase = bb * ROWS_PER_B
            rbase = lbase + NPAIR
            row0 = idxv[b, pl.ds(0, 16)]
            row1 = idxv[b, pl.ds(10, 16)]
            for i in range(NF - 1):
                sp = jnp.broadcast_to(row0[i] if i < 16 else row1[i - 10], (16,))
                ib[pl.ds(lbase + _S[i], 16)] = clv[i, pl.ds(0, 16)] + sp
                if 25 - i > 16:
                    ib[pl.ds(lbase + _S[i] + 16, 16)] = clv[i, pl.ds(16, 16)] + sp
            for i in range(NF - 2, -1, -1):
                off_i = i * V
                if i >= 10:
                    v = idxv[b, pl.ds(10, 16)] + off_i
                    ib[pl.ds(rbase + _S[i] + 9 - i, 16)] = v
                elif i == 9:
                    ib[pl.ds(rbase + _S[9], 16)] = idxv[b, pl.ds(10, 16)] + off_i
                else:
                    ib[pl.ds(rbase + _S[i], 16)] = idxv[b, pl.ds(i + 1, 16)] + off_i
                    ib[pl.ds(rbase + _S[i] + 9 - i, 16)] = idxv[b, pl.ds(10, 16)] + off_i
            fcib[pl.ds(bb * 32, 16)] = idxv[b, pl.ds(0, 16)]
            fcib[pl.ds(bb * 32 + 16, 16)] = idxv[b, pl.ds(10, 16)]

        descs = []
        for seg in range(NSEG):
            descs.append(pltpu.async_copy(
                w_hbm.at[ib.at[pl.ds(seg * SEG, SEG)]],
                rows.at[pl.ds(seg * SEG, SEG), :], sem))
        descs.append(pltpu.async_copy(fc_hbm.at[fcib], fcv, sem))
        for d_ in descs:
            d_.wait()

        for bb in range(NB):
            lbase = bb * ROWS_PER_B
            rbase = lbase + NPAIR

            def fma(t, acc, _lb=lbase, _rb=rbase):
                return acc + rows[_lb + t] * rows[_rb + t]

            acc = lax.fori_loop(0, NPAIR, fma, jnp.zeros((16,), jnp.float32))
            v0 = fcv[pl.ds(bb * 32, 16)]
            v1 = jnp.where(fc_mask, fcv[pl.ds(bb * 32 + 16, 16)], 0.0)
            bv = jnp.where(bias_mask, biasv[...], 0.0)
            z = allsum(acc + v0 + v1 + bv)
            zoff = (c // 4) * 16
            pos = (c % 4) * NB + bb
            zvec = zbuf[pl.ds(zoff, 16)]
            zbuf[pl.ds(zoff, 16)] = jnp.where(lane == pos, z, zvec)

    for k in range(BPW // 16):
        zv = zbuf[pl.ds(k * 16, 16)]
        zbuf[pl.ds(k * 16, 16)] = 1.0 / (1.0 + jnp.exp(-zv))
    pltpu.sync_copy(zbuf, out_hbm.at[pl.ds(base_b, BPW)])


@jax.jit
def _ffm(idx, w_flat, fc_flat, bias16, cl):
    mesh = plsc.VectorSubcoreMesh(core_axis_name="c", subcore_axis_name="s",
                                  num_cores=NC, num_subcores=NS)
    return pl.kernel(
        _body,
        out_type=jax.ShapeDtypeStruct((B,), jnp.float32),
        mesh=mesh,
        compiler_params=pltpu.CompilerParams(needs_layout_passes=False,
                                             use_tc_tiling_on_sc=False),
        scratch_types=[
            pltpu.VMEM((BPW, NF), jnp.int32),
            pltpu.VMEM((NF, 32), jnp.int32),
            pltpu.VMEM((ROWS_PAD,), jnp.int32),
            pltpu.VMEM((ROWS_PAD, D), jnp.float32),
            pltpu.VMEM((NB * 32,), jnp.int32),
            pltpu.VMEM((NB * 32,), jnp.float32),
            pltpu.VMEM((16,), jnp.float32),
            pltpu.VMEM((BPW,), jnp.float32),
            pltpu.SemaphoreType.DMA,
        ],
    )(idx, w_flat, fc_flat, bias16, cl)


def kernel(x, fc, bias, W):
    offsets = jnp.asarray(np.arange(NF, dtype=np.int32) * _FIELD_DIM)
    idx = x.astype(jnp.int32) + offsets[None, :]
    w_flat = W.reshape(NF * V, D)
    fc_flat = fc.reshape(V)
    bias16 = jnp.broadcast_to(bias.astype(jnp.float32), (16,))
    cl = jnp.asarray(_CL)
    return _ffm(idx, w_flat, fc_flat, bias16, cl)


# speedup vs baseline: 12.2100x; 1.2071x over previous
import jax
import jax.numpy as jnp
import numpy as np
from jax import lax
from jax.experimental import pallas as pl
from jax.experimental.pallas import tpu as pltpu
from jax.experimental.pallas import tpu_sc as plsc

_FIELD_DIM = 4000
NF = 26
D = 16
V = NF * _FIELD_DIM
B = 4096
NC, NS = 2, 16
NW = NC * NS
BPW = B // NW
NB = 4
NCHUNK = BPW // NB
NIDX = NB * NF
NROW = NF * NIDX


def _body(idx_hbm, w_hbm, fc_hbm, bias_hbm, out_hbm,
          idxv, ibx, rows, fcv, biasv, zbuf, sem):
    cid = lax.axis_index("c")
    sid = lax.axis_index("s")
    wid = sid * NC + cid
    base_b = wid * BPW

    pltpu.sync_copy(idx_hbm.at[pl.ds(base_b, BPW), :], idxv)
    pltpu.sync_copy(bias_hbm, biasv)

    lane = lax.iota(jnp.int32, 16)
    fc_mask = lane >= 6
    bias_mask = lane == 0

    @pl.loop(0, NCHUNK)
    def _chunk(c):
        for bb in range(NB):
            b = c * NB + bb
            ibx[pl.ds(bb * NF, 16)] = idxv[b, pl.ds(0, 16)]
            ibx[pl.ds(bb * NF + 10, 16)] = idxv[b, pl.ds(10, 16)]

        descs = []
        for t in range(NF):
            descs.append(pltpu.async_copy(
                w_hbm.at[t].at[ibx],
                rows.at[pl.ds(t * NIDX, NIDX), :], sem))
        descs.append(pltpu.async_copy(fc_hbm.at[ibx], fcv, sem))
        for d_ in descs:
            d_.wait()

        for bb in range(NB):
            boff = bb * NF
            acc = jnp.zeros((16,), jnp.float32)
            for d in range(1, NF):
                @pl.loop(0, NF - d, init_carry=acc, unroll=4)
                def _diag(i, a, _d=d, _boff=boff):
                    return a + (rows[i * (NIDX + 1) + _d * NIDX + _boff]
                                * rows[i * (NIDX + 1) + _d + _boff])
                acc = _diag
            v0 = fcv[pl.ds(boff, 16)]
            v1 = jnp.where(fc_mask, fcv[pl.ds(boff + 10, 16)], 0.0)
            bv = jnp.where(bias_mask, biasv[...], 0.0)
            z = jnp.sum(acc + v0 + v1 + bv)
            zoff = (c // 4) * 16
            pos = (c % 4) * NB + bb
            zvec = zbuf[pl.ds(zoff, 16)]
            zbuf[pl.ds(zoff, 16)] = jnp.where(lane == pos, z, zvec)

    for k in range(BPW // 16):
        zv = zbuf[pl.ds(k * 16, 16)]
        zbuf[pl.ds(k * 16, 16)] = 1.0 / (1.0 + jnp.exp(-zv))
    pltpu.sync_copy(zbuf, out_hbm.at[pl.ds(base_b, BPW)])


@jax.jit
def _ffm(idx, w, fc_flat, bias16):
    mesh = plsc.VectorSubcoreMesh(core_axis_name="c", subcore_axis_name="s",
                                  num_cores=NC, num_subcores=NS)
    return pl.kernel(
        _body,
        out_type=jax.ShapeDtypeStruct((B,), jnp.float32),
        mesh=mesh,
        compiler_params=pltpu.CompilerParams(needs_layout_passes=False,
                                             use_tc_tiling_on_sc=False),
        scratch_types=[
            pltpu.VMEM((BPW, NF), jnp.int32),
            pltpu.VMEM((NIDX,), jnp.int32),
            pltpu.VMEM((NROW, D), jnp.float32),
            pltpu.VMEM((NIDX,), jnp.float32),
            pltpu.VMEM((16,), jnp.float32),
            pltpu.VMEM((BPW,), jnp.float32),
            pltpu.SemaphoreType.DMA,
        ],
    )(idx, w, fc_flat, bias16)


def kernel(x, fc, bias, W):
    offsets = jnp.asarray(np.arange(NF, dtype=np.int32) * _FIELD_DIM)
    idx = x.astype(jnp.int32) + offsets[None, :]
    fc_flat = fc.reshape(V)
    bias16 = jnp.broadcast_to(bias.astype(jnp.float32), (16,))
    return _ffm(idx, W, fc_flat, bias16)


# speedup vs baseline: 12.9480x; 1.0604x over previous
import jax
import jax.numpy as jnp
import numpy as np
from jax import lax
from jax.experimental import pallas as pl
from jax.experimental.pallas import tpu as pltpu
from jax.experimental.pallas import tpu_sc as plsc

_FIELD_DIM = 4000
NF = 26
D = 16
V = NF * _FIELD_DIM
B = 4096
NC, NS = 2, 16
NW = NC * NS
BPW = B // NW
NB = 4
NCHUNK = BPW // NB
NIDX = NB * NF
NROW = NF * NIDX


def _body(idx_hbm, w_hbm, fc_hbm, bias_hbm, out_hbm,
          idxv, ibx0, ibx1, rows0, rows1, fcv0, fcv1, biasv, zbuf,
          sem0, sem1):
    cid = lax.axis_index("c")
    sid = lax.axis_index("s")
    wid = sid * NC + cid
    base_b = wid * BPW

    pltpu.sync_copy(idx_hbm.at[pl.ds(base_b, BPW), :], idxv)
    pltpu.sync_copy(bias_hbm, biasv)

    lane = lax.iota(jnp.int32, 16)
    fc_mask = lane >= 6
    bias_mask = lane == 0

    def build(c, ibx):
        for bb in range(NB):
            b = c * NB + bb
            ibx[pl.ds(bb * NF, 16)] = idxv[b, pl.ds(0, 16)]
            ibx[pl.ds(bb * NF + 10, 16)] = idxv[b, pl.ds(10, 16)]

    def fire(ibx, rows, fcv, sem):
        for t in range(NF):
            pltpu.async_copy(w_hbm.at[t].at[ibx],
                             rows.at[pl.ds(t * NIDX, NIDX), :], sem)
        pltpu.async_copy(fc_hbm.at[ibx], fcv, sem)

    def drain(rows, fcv, sem):
        pltpu.make_async_copy(w_hbm.at[0].at[pl.ds(0, NROW), :], rows, sem).wait()
        pltpu.make_async_copy(fc_hbm.at[pl.ds(0, NIDX)], fcv, sem).wait()

    def compute(c, rows, fcv):
        for bb in range(NB):
            boff = bb * NF
            acc = jnp.zeros((16,), jnp.float32)
            for d in range(1, NF):
                @pl.loop(0, NF - d, init_carry=acc, unroll=4)
                def _diag(i, a, _d=d, _boff=boff):
                    return a + (rows[i * (NIDX + 1) + _d * NIDX + _boff]
                                * rows[i * (NIDX + 1) + _d + _boff])
                acc = _diag
            v0 = fcv[pl.ds(boff, 16)]
            v1 = jnp.where(fc_mask, fcv[pl.ds(boff + 10, 16)], 0.0)
            bv = jnp.where(bias_mask, biasv[...], 0.0)
            z = jnp.sum(acc + v0 + v1 + bv)
            zoff = (c // 4) * 16
            pos = (c % 4) * NB + bb
            zvec = zbuf[pl.ds(zoff, 16)]
            zbuf[pl.ds(zoff, 16)] = jnp.where(lane == pos, z, zvec)

    build(0, ibx0)
    fire(ibx0, rows0, fcv0, sem0)

    @pl.loop(0, NCHUNK // 2)
    def _pair(cp):
        c0 = cp * 2
        c1 = c0 + 1
        build(c1, ibx1)
        fire(ibx1, rows1, fcv1, sem1)
        drain(rows0, fcv0, sem0)
        compute(c0, rows0, fcv0)

        @pl.when(c1 + 1 < NCHUNK)
        def _prefetch():
            build(c1 + 1, ibx0)
            fire(ibx0, rows0, fcv0, sem0)

        drain(rows1, fcv1, sem1)
        compute(c1, rows1, fcv1)

    for k in range(BPW // 16):
        zv = zbuf[pl.ds(k * 16, 16)]
        zbuf[pl.ds(k * 16, 16)] = 1.0 / (1.0 + jnp.exp(-zv))
    pltpu.sync_copy(zbuf, out_hbm.at[pl.ds(base_b, BPW)])


@jax.jit
def _ffm(idx, w, fc_flat, bias16):
    mesh = plsc.VectorSubcoreMesh(core_axis_name="c", subcore_axis_name="s",
                                  num_cores=NC, num_subcores=NS)
    return pl.kernel(
        _body,
        out_type=jax.ShapeDtypeStruct((B,), jnp.float32),
        mesh=mesh,
        compiler_params=pltpu.CompilerParams(needs_layout_passes=False,
                                             use_tc_tiling_on_sc=False),
        scratch_types=[
            pltpu.VMEM((BPW, NF), jnp.int32),
            pltpu.VMEM((NIDX,), jnp.int32),
            pltpu.VMEM((NIDX,), jnp.int32),
            pltpu.VMEM((NROW, D), jnp.float32),
            pltpu.VMEM((NROW, D), jnp.float32),
            pltpu.VMEM((NIDX,), jnp.float32),
            pltpu.VMEM((NIDX,), jnp.float32),
            pltpu.VMEM((16,), jnp.float32),
            pltpu.VMEM((BPW,), jnp.float32),
            pltpu.SemaphoreType.DMA,
            pltpu.SemaphoreType.DMA,
        ],
    )(idx, w, fc_flat, bias16)


def kernel(x, fc, bias, W):
    offsets = jnp.asarray(np.arange(NF, dtype=np.int32) * _FIELD_DIM)
    idx = x.astype(jnp.int32) + offsets[None, :]
    fc_flat = fc.reshape(V)
    bias16 = jnp.broadcast_to(bias.astype(jnp.float32), (16,))
    return _ffm(idx, W, fc_flat, bias16)
